# baseline (speedup 1.0000x reference)
import jax
import jax.numpy as jnp
from jax import lax
from jax.experimental import pallas as pl
from jax.experimental.pallas import tpu as pltpu
from jax.experimental.pallas import tpu_sc as plsc

D = 256
HID = 256
OUT = 64

NW = 32
CHUNK = 80
CHUNKS_PER_W = 5
ROWS_PER_W = CHUNK * CHUNKS_PER_W
N_HALF = NW * ROWS_PER_W
N_OUT = 25000
TC_BLOCK = 1600


def _sc_gather(imr_hbm, gr_hbm, idx_hbm, x_hbm,
               idx_v, b1a, b1b, b2a, b2b, o1a, o1b, o2a, o2b,
               sem_g, sem_wa, sem_wb):
    wid = lax.axis_index("s") * 2 + lax.axis_index("c")
    base = wid * ROWS_PER_W
    bufs = ((b1a, b1b), (b2a, b2b))
    obufs = ((o1a, o1b), (o2a, o2b))
    sem_w = (sem_wa, sem_wb)
    pltpu.sync_copy(idx_hbm.at[pl.ds(base, ROWS_PER_W)], idx_v)

    def pack_chunk(fbuf, obuf):
        @plsc.parallel_loop(0, CHUNK // 2, 1, unroll=2)
        def pair_body(k):
            for t in range(0, D, 16):
                a = fbuf[2 * k, pl.ds(t, 16)]
                b = fbuf[2 * k + 1, pl.ds(t, 16)]
                p = plsc.pack(a, b, format=plsc.PackFormat.INTERLEAVED)
                obuf[k, pl.ds(t, 16)] = plsc.bitcast(p, jnp.int32)

    idx0 = idx_v.at[pl.ds(0, CHUNK)]
    gcur = [pltpu.async_copy(imr_hbm.at[idx0], bufs[0][0], sem_g),
            pltpu.async_copy(gr_hbm.at[idx0], bufs[1][0], sem_g)]
    pending = []
    for c in range(CHUNKS_PER_W):
        cur = c % 2
        nxt = (c + 1) % 2
        for d in gcur:
            d.wait()
        if c + 1 < CHUNKS_PER_W:
            idx_c = idx_v.at[pl.ds((c + 1) * CHUNK, CHUNK)]
            gcur = [pltpu.async_copy(imr_hbm.at[idx_c], bufs[0][nxt], sem_g),
                    pltpu.async_copy(gr_hbm.at[idx_c], bufs[1][nxt], sem_g)]
        if len(pending) >= 2:
            for d in pending.pop(0):
                d.wait()
        pack_chunk(bufs[0][cur], obufs[0][cur])
        pack_chunk(bufs[1][cur], obufs[1][cur])
        rw0 = wid * (ROWS_PER_W // 2) + c * (CHUNK // 2)
        pending.append([
            pltpu.async_copy(
                obufs[0][cur],
                x_hbm.at[pl.ds(rw0, CHUNK // 2), pl.ds(0, D)],
                sem_w[cur]),
            pltpu.async_copy(
                obufs[1][cur],
                x_hbm.at[pl.ds(rw0, CHUNK // 2), pl.ds(D, D)],
                sem_w[cur]),
        ])
    for grp in pending:
        for d in grp:
            d.wait()


def _gather_rows(imr, gr, idx_half):
    mesh = plsc.VectorSubcoreMesh(core_axis_name="c", subcore_axis_name="s")
    f = pl.kernel(
        _sc_gather,
        out_type=jax.ShapeDtypeStruct((N_HALF // 2, 2 * D), jnp.int32),
        mesh=mesh,
        scratch_types=[
            pltpu.VMEM((ROWS_PER_W,), jnp.int32),
            pltpu.VMEM((CHUNK, D), jnp.float32),
            pltpu.VMEM((CHUNK, D), jnp.float32),
            pltpu.VMEM((CHUNK, D), jnp.float32),
            pltpu.VMEM((CHUNK, D), jnp.float32),
            pltpu.VMEM((CHUNK // 2, D), jnp.int32),
            pltpu.VMEM((CHUNK // 2, D), jnp.int32),
            pltpu.VMEM((CHUNK // 2, D), jnp.int32),
            pltpu.VMEM((CHUNK // 2, D), jnp.int32),
            pltpu.SemaphoreType.DMA,
            pltpu.SemaphoreType.DMA,
            pltpu.SemaphoreType.DMA,
        ],
        compiler_params=pltpu.CompilerParams(needs_layout_passes=False),
    )
    return f(imr, gr, idx_half)


def _mlp_body(x_ref, w1_ref, w2_ref, b1_ref, b2_ref, oprev_ref, o_ref):
    del oprev_ref
    x = pltpu.bitcast(x_ref[...], jnp.bfloat16)
    h = jnp.dot(x, w1_ref[...], preferred_element_type=jnp.float32)
    h = jnp.maximum(h + b1_ref[...], 0.0)
    o_ref[...] = (
        jnp.dot(h, w2_ref[...], preferred_element_type=jnp.float32)
        + b2_ref[...]
    )


def _mlp(x, w1, w2, b1r, b2r, o_prev, half):
    off = half * (N_HALF // TC_BLOCK)
    return pl.pallas_call(
        _mlp_body,
        grid=(N_HALF // TC_BLOCK,),
        in_specs=[
            pl.BlockSpec((TC_BLOCK // 2, 2 * D), lambda i: (i, 0)),
            pl.BlockSpec((2 * D, HID), lambda i: (0, 0)),
            pl.BlockSpec((HID, OUT), lambda i: (0, 0)),
            pl.BlockSpec((1, HID), lambda i: (0, 0)),
            pl.BlockSpec((1, OUT), lambda i: (0, 0)),
            pl.BlockSpec((TC_BLOCK, OUT), lambda i: (i + off, 0)),
        ],
        out_specs=pl.BlockSpec((TC_BLOCK, OUT), lambda i: (i + off, 0)),
        out_shape=jax.ShapeDtypeStruct((N_OUT, OUT), jnp.float32),
        input_output_aliases={5: 0},
    )(x, w1, w2, b1r, b2r, o_prev)


def kernel(input_molecule_representations, graph_representations,
           graphs_requiring_node_choices, W1, b1, W2, b2):
    n_sel = graphs_requiring_node_choices.shape[0]
    idx = graphs_requiring_node_choices.astype(jnp.int32)
    idx_pad = jnp.concatenate(
        [idx, jnp.zeros((2 * N_HALF - n_sel,), jnp.int32)])
    w1 = W1.astype(jnp.bfloat16)
    b1r = b1.reshape(1, HID)
    b2r = b2.reshape(1, OUT)
    out = jnp.zeros((N_OUT, OUT), jnp.float32)
    for h in range(2):
        x = _gather_rows(
            input_molecule_representations, graph_representations,
            idx_pad[h * N_HALF:(h + 1) * N_HALF])
        out = _mlp(x, w1, W2, b1r, b2r, out, h)
    return out


# speedup vs baseline: 1.3385x; 1.3385x over previous
import jax
import jax.numpy as jnp
from jax import lax
from jax.experimental import pallas as pl
from jax.experimental.pallas import tpu as pltpu
from jax.experimental.pallas import tpu_sc as plsc

D = 256
HID = 256
OUT = 64

NW = 32
CHUNK = 56
CHUNKS_PER_W = 7
ROWS_PER_W = CHUNK * CHUNKS_PER_W
N_HALF = NW * ROWS_PER_W
N_OUT = 25000
TC_BLOCK = 1568


def _sc_gather(imr_hbm, gr_hbm, idx_hbm, x_hbm,
               idx_v, b1a, b1b, b2a, b2b, sem_g, sem_wa, sem_wb):
    wid = lax.axis_index("s") * 2 + lax.axis_index("c")
    base = wid * ROWS_PER_W
    bufs1 = (b1a, b1b)
    bufs2 = (b2a, b2b)
    sem_w = (sem_wa, sem_wb)
    pltpu.sync_copy(idx_hbm.at[pl.ds(base, ROWS_PER_W)], idx_v)
    idx0 = idx_v.at[pl.ds(0, CHUNK)]
    gcur = [pltpu.async_copy(imr_hbm.at[idx0], bufs1[0], sem_g),
            pltpu.async_copy(gr_hbm.at[idx0], bufs2[0], sem_g)]
    pending = []
    for c in range(CHUNKS_PER_W):
        cur = c % 2
        nxt = (c + 1) % 2
        for d in gcur:
            d.wait()
        row0 = base + c * CHUNK
        pending.append([
            pltpu.async_copy(
                bufs1[cur], x_hbm.at[pl.ds(row0, CHUNK), pl.ds(0, D)],
                sem_w[cur]),
            pltpu.async_copy(
                bufs2[cur], x_hbm.at[pl.ds(row0, CHUNK), pl.ds(D, D)],
                sem_w[cur]),
        ])
        if c + 1 < CHUNKS_PER_W:
            if len(pending) >= 2:
                for d in pending.pop(0):
                    d.wait()
            idx_c = idx_v.at[pl.ds((c + 1) * CHUNK, CHUNK)]
            gcur = [pltpu.async_copy(imr_hbm.at[idx_c], bufs1[nxt], sem_g),
                    pltpu.async_copy(gr_hbm.at[idx_c], bufs2[nxt], sem_g)]
    for grp in pending:
        for d in grp:
            d.wait()


def _gather_rows(imr, gr, idx_half):
    mesh = plsc.VectorSubcoreMesh(core_axis_name="c", subcore_axis_name="s")
    f = pl.kernel(
        _sc_gather,
        out_type=jax.ShapeDtypeStruct((N_HALF, 2 * D), jnp.float32),
        mesh=mesh,
        scratch_types=[
            pltpu.VMEM((ROWS_PER_W,), jnp.int32),
            pltpu.VMEM((CHUNK, D), jnp.float32),
            pltpu.VMEM((CHUNK, D), jnp.float32),
            pltpu.VMEM((CHUNK, D), jnp.float32),
            pltpu.VMEM((CHUNK, D), jnp.float32),
            pltpu.SemaphoreType.DMA,
            pltpu.SemaphoreType.DMA,
            pltpu.SemaphoreType.DMA,
        ],
    )
    return f(imr, gr, idx_half)


def _mlp_body(*refs):
    x_ref, w1_ref, w2_ref, b1_ref, b2_ref = refs[:5]
    o_ref = refs[-1]
    h = jnp.dot(x_ref[...], w1_ref[...], preferred_element_type=jnp.float32)
    h = jnp.maximum(h + b1_ref[...], 0.0)
    o_ref[...] = (
        jnp.dot(h, w2_ref[...], preferred_element_type=jnp.float32)
        + b2_ref[...]
    )


def _mlp(x, w1, w2, b1r, b2r, o_prev, half):
    off = half * (N_HALF // TC_BLOCK)
    in_specs = [
        pl.BlockSpec((TC_BLOCK, 2 * D), lambda i: (i, 0)),
        pl.BlockSpec((2 * D, HID), lambda i: (0, 0)),
        pl.BlockSpec((HID, OUT), lambda i: (0, 0)),
        pl.BlockSpec((1, HID), lambda i: (0, 0)),
        pl.BlockSpec((1, OUT), lambda i: (0, 0)),
    ]
    args = [x, w1, w2, b1r, b2r]
    aliases = {}
    if half:
        in_specs.append(pl.BlockSpec((TC_BLOCK, OUT), lambda i: (i + off, 0)))
        args.append(o_prev)
        aliases = {5: 0}
    return pl.pallas_call(
        _mlp_body,
        grid=(N_HALF // TC_BLOCK,),
        in_specs=in_specs,
        out_specs=pl.BlockSpec((TC_BLOCK, OUT), lambda i: (i + off, 0)),
        out_shape=jax.ShapeDtypeStruct((N_OUT, OUT), jnp.float32),
        input_output_aliases=aliases,
    )(*args)


def kernel(input_molecule_representations, graph_representations,
           graphs_requiring_node_choices, W1, b1, W2, b2):
    n_sel = graphs_requiring_node_choices.shape[0]
    idx = graphs_requiring_node_choices.astype(jnp.int32)
    idx_pad = jnp.concatenate(
        [idx, jnp.zeros((2 * N_HALF - n_sel,), jnp.int32)])
    b1r = b1.reshape(1, HID)
    b2r = b2.reshape(1, OUT)
    out = None
    for h in range(2):
        x = _gather_rows(
            input_molecule_representations, graph_representations,
            idx_pad[h * N_HALF:(h + 1) * N_HALF])
        out = _mlp(x, W1, W2, b1r, b2r, out, h)
    return out


# speedup vs baseline: 1.3534x; 1.0112x over previous
import jax
import jax.numpy as jnp
from jax import lax
from jax.experimental import pallas as pl
from jax.experimental.pallas import tpu as pltpu
from jax.experimental.pallas import tpu_sc as plsc

D = 256
HID = 256
OUT = 64

NW = 32
CHUNK = 56
SPLIT_CHUNKS = (8, 6)
N_PARTS = (14336, 10752)
N_OUT = 25000
TC_BLOCK = 1792


def _make_sc_gather(chunks_per_w):
  rows_per_w = CHUNK * chunks_per_w

  def _sc_gather(imr_hbm, gr_hbm, idx_hbm, x_hbm,
                 idx_v, b1a, b1b, b2a, b2b, sem_g, sem_wa, sem_wb):
    wid = lax.axis_index("s") * 2 + lax.axis_index("c")
    base = wid * rows_per_w
    bufs1 = (b1a, b1b)
    bufs2 = (b2a, b2b)
    sem_w = (sem_wa, sem_wb)
    pltpu.sync_copy(idx_hbm.at[pl.ds(base, rows_per_w)], idx_v)
    idx0 = idx_v.at[pl.ds(0, CHUNK)]
    gcur = [pltpu.async_copy(imr_hbm.at[idx0], bufs1[0], sem_g),
            pltpu.async_copy(gr_hbm.at[idx0], bufs2[0], sem_g)]
    pending = []
    for c in range(chunks_per_w):
        cur = c % 2
        nxt = (c + 1) % 2
        for d in gcur:
            d.wait()
        row0 = base + c * CHUNK
        pending.append([
            pltpu.async_copy(
                bufs1[cur], x_hbm.at[pl.ds(row0, CHUNK), pl.ds(0, D)],
                sem_w[cur]),
            pltpu.async_copy(
                bufs2[cur], x_hbm.at[pl.ds(row0, CHUNK), pl.ds(D, D)],
                sem_w[cur]),
        ])
        if c + 1 < chunks_per_w:
            if len(pending) >= 2:
                for d in pending.pop(0):
                    d.wait()
            idx_c = idx_v.at[pl.ds((c + 1) * CHUNK, CHUNK)]
            gcur = [pltpu.async_copy(imr_hbm.at[idx_c], bufs1[nxt], sem_g),
                    pltpu.async_copy(gr_hbm.at[idx_c], bufs2[nxt], sem_g)]
    for grp in pending:
        for d in grp:
            d.wait()

  return _sc_gather


def _gather_rows(imr, gr, idx_part, chunks_per_w):
    mesh = plsc.VectorSubcoreMesh(core_axis_name="c", subcore_axis_name="s")
    f = pl.kernel(
        _make_sc_gather(chunks_per_w),
        out_type=jax.ShapeDtypeStruct(
            (NW * CHUNK * chunks_per_w, 2 * D), jnp.float32),
        mesh=mesh,
        scratch_types=[
            pltpu.VMEM((CHUNK * chunks_per_w,), jnp.int32),
            pltpu.VMEM((CHUNK, D), jnp.float32),
            pltpu.VMEM((CHUNK, D), jnp.float32),
            pltpu.VMEM((CHUNK, D), jnp.float32),
            pltpu.VMEM((CHUNK, D), jnp.float32),
            pltpu.SemaphoreType.DMA,
            pltpu.SemaphoreType.DMA,
            pltpu.SemaphoreType.DMA,
        ],
    )
    return f(imr, gr, idx_part)


def _mlp_body(*refs):
    x_ref, w1_ref, w2_ref, b1_ref, b2_ref = refs[:5]
    o_ref = refs[-1]
    h = jnp.dot(x_ref[...], w1_ref[...], preferred_element_type=jnp.float32)
    h = jnp.maximum(h + b1_ref[...], 0.0)
    o_ref[...] = (
        jnp.dot(h, w2_ref[...], preferred_element_type=jnp.float32)
        + b2_ref[...]
    )


def _mlp(x, w1, w2, b1r, b2r, o_prev, half):
    off = half * (N_PARTS[0] // TC_BLOCK)
    in_specs = [
        pl.BlockSpec((TC_BLOCK, 2 * D), lambda i: (i, 0)),
        pl.BlockSpec((2 * D, HID), lambda i: (0, 0)),
        pl.BlockSpec((HID, OUT), lambda i: (0, 0)),
        pl.BlockSpec((1, HID), lambda i: (0, 0)),
        pl.BlockSpec((1, OUT), lambda i: (0, 0)),
    ]
    args = [x, w1, w2, b1r, b2r]
    aliases = {}
    if half:
        in_specs.append(pl.BlockSpec((TC_BLOCK, OUT), lambda i: (i + off, 0)))
        args.append(o_prev)
        aliases = {5: 0}
    return pl.pallas_call(
        _mlp_body,
        grid=(N_PARTS[half] // TC_BLOCK,),
        in_specs=in_specs,
        out_specs=pl.BlockSpec((TC_BLOCK, OUT), lambda i: (i + off, 0)),
        out_shape=jax.ShapeDtypeStruct((N_OUT, OUT), jnp.float32),
        input_output_aliases=aliases,
    )(*args)


def kernel(input_molecule_representations, graph_representations,
           graphs_requiring_node_choices, W1, b1, W2, b2):
    n_sel = graphs_requiring_node_choices.shape[0]
    idx = graphs_requiring_node_choices.astype(jnp.int32)
    idx_pad = jnp.concatenate(
        [idx, jnp.zeros((sum(N_PARTS) - n_sel,), jnp.int32)])
    b1r = b1.reshape(1, HID)
    b2r = b2.reshape(1, OUT)
    out = None
    row = 0
    for h in range(2):
        x = _gather_rows(
            input_molecule_representations, graph_representations,
            idx_pad[row:row + N_PARTS[h]], SPLIT_CHUNKS[h])
        out = _mlp(x, W1, W2, b1r, b2r, out, h)
        row += N_PARTS[h]
    return out


# speedup vs baseline: 1.5405x; 1.1382x over previous
import jax
import jax.numpy as jnp
from jax import lax
from jax.experimental import pallas as pl
from jax.experimental.pallas import tpu as pltpu
from jax.experimental.pallas import tpu_sc as plsc

D = 256
HID = 256
OUT = 64

NW = 32
CHUNK = 56
SPLIT_CHUNKS = (8, 6)
N_PARTS = (14336, 10752)
N_OUT = 25000
TC_BLOCK = 1792


def _make_sc_gather(chunks_per_w):
  rows_per_w = CHUNK * chunks_per_w

  def _sc_gather(imr_hbm, gr_hbm, idx_hbm, x_hbm,
                 idx_v, b1a, b1b, b2a, b2b, sem_g, sem_wa, sem_wb):
    wid = lax.axis_index("s") * 2 + lax.axis_index("c")
    base = wid * rows_per_w
    bufs1 = (b1a, b1b)
    bufs2 = (b2a, b2b)
    sem_w = (sem_wa, sem_wb)
    pltpu.sync_copy(idx_hbm.at[pl.ds(base, rows_per_w)], idx_v)
    idx0 = idx_v.at[pl.ds(0, CHUNK)]
    gcur = [pltpu.async_copy(imr_hbm.at[idx0], bufs1[0], sem_g),
            pltpu.async_copy(gr_hbm.at[idx0], bufs2[0], sem_g)]
    pending = []
    for c in range(chunks_per_w):
        cur = c % 2
        nxt = (c + 1) % 2
        for d in gcur:
            d.wait()
        row0 = base + c * CHUNK
        pending.append([
            pltpu.async_copy(
                bufs1[cur], x_hbm.at[pl.ds(row0, CHUNK), pl.ds(0, D)],
                sem_w[cur]),
            pltpu.async_copy(
                bufs2[cur], x_hbm.at[pl.ds(row0, CHUNK), pl.ds(D, D)],
                sem_w[cur]),
        ])
        if c + 1 < chunks_per_w:
            if len(pending) >= 2:
                for d in pending.pop(0):
                    d.wait()
            idx_c = idx_v.at[pl.ds((c + 1) * CHUNK, CHUNK)]
            gcur = [pltpu.async_copy(imr_hbm.at[idx_c], bufs1[nxt], sem_g),
                    pltpu.async_copy(gr_hbm.at[idx_c], bufs2[nxt], sem_g)]
    for grp in pending:
        for d in grp:
            d.wait()

  return _sc_gather


def _gather_rows(imr, gr, idx_part, chunks_per_w):
    mesh = plsc.VectorSubcoreMesh(core_axis_name="c", subcore_axis_name="s")
    f = pl.kernel(
        _make_sc_gather(chunks_per_w),
        out_type=jax.ShapeDtypeStruct(
            (NW * CHUNK * chunks_per_w, 2 * D), jnp.float32),
        mesh=mesh,
        scratch_types=[
            pltpu.VMEM((CHUNK * chunks_per_w,), jnp.int32),
            pltpu.VMEM((CHUNK, D), jnp.float32),
            pltpu.VMEM((CHUNK, D), jnp.float32),
            pltpu.VMEM((CHUNK, D), jnp.float32),
            pltpu.VMEM((CHUNK, D), jnp.float32),
            pltpu.SemaphoreType.DMA,
            pltpu.SemaphoreType.DMA,
            pltpu.SemaphoreType.DMA,
        ],
    )
    return f(imr, gr, idx_part)


def _mlp_body(*refs):
    x_ref, w1_ref, w2_ref, b1_ref, b2_ref = refs[:5]
    o_ref = refs[-1]
    h = jnp.dot(x_ref[...], w1_ref[...], preferred_element_type=jnp.float32)
    h = jnp.maximum(h + b1_ref[...], 0.0)
    o_t = jax.lax.dot_general(
        w2_ref[...], h, (((0,), (1,)), ((), ())),
        preferred_element_type=jnp.float32)
    o_ref[...] = o_t + b2_ref[...]


def _mlp(x, w1, w2, b1r, b2r, o_prev, half):
    off = half * (N_PARTS[0] // TC_BLOCK)
    in_specs = [
        pl.BlockSpec((TC_BLOCK, 2 * D), lambda i: (i, 0)),
        pl.BlockSpec((2 * D, HID), lambda i: (0, 0)),
        pl.BlockSpec((HID, OUT), lambda i: (0, 0)),
        pl.BlockSpec((1, HID), lambda i: (0, 0)),
        pl.BlockSpec((OUT, 1), lambda i: (0, 0)),
    ]
    args = [x, w1, w2, b1r, b2r]
    aliases = {}
    if half:
        in_specs.append(pl.BlockSpec((OUT, TC_BLOCK), lambda i: (0, i + off)))
        args.append(o_prev)
        aliases = {5: 0}
    return pl.pallas_call(
        _mlp_body,
        grid=(N_PARTS[half] // TC_BLOCK,),
        in_specs=in_specs,
        out_specs=pl.BlockSpec((OUT, TC_BLOCK), lambda i: (0, i + off)),
        out_shape=jax.ShapeDtypeStruct((OUT, N_OUT), jnp.float32),
        input_output_aliases=aliases,
    )(*args)


def kernel(input_molecule_representations, graph_representations,
           graphs_requiring_node_choices, W1, b1, W2, b2):
    n_sel = graphs_requiring_node_choices.shape[0]
    idx = graphs_requiring_node_choices.astype(jnp.int32)
    idx_pad = jnp.concatenate(
        [idx, jnp.zeros((sum(N_PARTS) - n_sel,), jnp.int32)])
    b1r = b1.reshape(1, HID)
    b2r = b2.reshape(OUT, 1)
    out = None
    row = 0
    for h in range(2):
        x = _gather_rows(
            input_molecule_representations, graph_representations,
            idx_pad[row:row + N_PARTS[h]], SPLIT_CHUNKS[h])
        out = _mlp(x, W1, W2, b1r, b2r, out, h)
        row += N_PARTS[h]
    return out.T
